# baseline (speedup 1.0000x reference)
import jax
import jax.numpy as jnp
from jax.experimental import pallas as pl

_MOVIE_ROWS = 2000


def _mm_kernel(xa_ref, xb_ref, w_ref, b_ref, oa_ref, ob_ref):
    oa_ref[...] = (
        jnp.dot(xa_ref[0], w_ref[...], preferred_element_type=jnp.float32)
        + b_ref[...]
    )
    ob_ref[...] = (
        jnp.dot(xb_ref[0], w_ref[...], preferred_element_type=jnp.float32)
        + b_ref[...]
    )


def kernel(movie_x, user_emb_weight, W, b):
    m, k = movie_x.shape
    n = W.shape[1]
    half = m // 2
    x3 = movie_x.reshape(2, half, k)
    top, bot = pl.pallas_call(
        _mm_kernel,
        grid=(half // _MOVIE_ROWS,),
        in_specs=[
            pl.BlockSpec((1, _MOVIE_ROWS, k), lambda i: (0, i, 0)),
            pl.BlockSpec((1, _MOVIE_ROWS, k), lambda i: (1, i, 0)),
            pl.BlockSpec((k, n), lambda i: (0, 0)),
            pl.BlockSpec((n,), lambda i: (0,)),
        ],
        out_specs=[
            pl.BlockSpec((_MOVIE_ROWS, n), lambda i: (i, 0)),
            pl.BlockSpec((_MOVIE_ROWS, n), lambda i: (i, 0)),
        ],
        out_shape=[
            jax.ShapeDtypeStruct((half, n), jnp.float32),
            jax.ShapeDtypeStruct((half, n), jnp.float32),
        ],
    )(x3, x3, W, b)
    return (user_emb_weight, jnp.concatenate([top, bot], axis=0))


# speedup vs baseline: 1.1458x; 1.1458x over previous
import jax
import jax.numpy as jnp
from jax.experimental import pallas as pl

_MOVIE_ROWS = 5000


def _mm_kernel(xa_ref, xb_ref, w_ref, b_ref, o_ref):
    o_ref[0] = (
        jnp.dot(xa_ref[0], w_ref[...], preferred_element_type=jnp.float32)
        + b_ref[...]
    )
    o_ref[1] = (
        jnp.dot(xb_ref[0], w_ref[...], preferred_element_type=jnp.float32)
        + b_ref[...]
    )


def kernel(movie_x, user_emb_weight, W, b):
    m, k = movie_x.shape
    n = W.shape[1]
    half = m // 2
    x3 = movie_x.reshape(2, half, k)
    movie3 = pl.pallas_call(
        _mm_kernel,
        grid=(half // _MOVIE_ROWS,),
        in_specs=[
            pl.BlockSpec((1, _MOVIE_ROWS, k), lambda i: (0, i, 0)),
            pl.BlockSpec((1, _MOVIE_ROWS, k), lambda i: (1, i, 0)),
            pl.BlockSpec((k, n), lambda i: (0, 0)),
            pl.BlockSpec((n,), lambda i: (0,)),
        ],
        out_specs=pl.BlockSpec((2, _MOVIE_ROWS, n), lambda i: (0, i, 0)),
        out_shape=jax.ShapeDtypeStruct((2, half, n), jnp.float32),
    )(x3, x3, W, b)
    return (user_emb_weight, movie3.reshape(m, n))


# speedup vs baseline: 1.1612x; 1.0134x over previous
import jax
import jax.numpy as jnp
from jax.experimental import pallas as pl

_MOVIE_ROWS = 10000


def _mm_kernel(xa_ref, xb_ref, w_ref, b_ref, o_ref):
    o_ref[0] = (
        jnp.dot(xa_ref[0], w_ref[...], preferred_element_type=jnp.float32)
        + b_ref[...]
    )
    o_ref[1] = (
        jnp.dot(xb_ref[0], w_ref[...], preferred_element_type=jnp.float32)
        + b_ref[...]
    )


def kernel(movie_x, user_emb_weight, W, b):
    m, k = movie_x.shape
    n = W.shape[1]
    half = m // 2
    x3 = movie_x.reshape(2, half, k)
    movie3 = pl.pallas_call(
        _mm_kernel,
        grid=(half // _MOVIE_ROWS,),
        in_specs=[
            pl.BlockSpec((1, _MOVIE_ROWS, k), lambda i: (0, i, 0)),
            pl.BlockSpec((1, _MOVIE_ROWS, k), lambda i: (1, i, 0)),
            pl.BlockSpec((k, n), lambda i: (0, 0)),
            pl.BlockSpec((n,), lambda i: (0,)),
        ],
        out_specs=pl.BlockSpec((2, _MOVIE_ROWS, n), lambda i: (0, i, 0)),
        out_shape=jax.ShapeDtypeStruct((2, half, n), jnp.float32),
    )(x3, x3, W, b)
    return (user_emb_weight, movie3.reshape(m, n))
